# baseline (speedup 1.0000x reference)
import jax
import jax.numpy as jnp
from jax import lax
from jax.experimental import pallas as pl
from jax.experimental.pallas import tpu as pltpu
from jax.experimental.pallas import tpu_sc as plsc

B, N, M = 4, 4096, 4096
NW = 32
WPB = NW // B
CH = N // WPB
L = 16
NG = CH // L
NC = M // L


def _round_bf16(v):
    u = lax.bitcast_convert_type(v, jnp.int32)
    lsb = lax.shift_right_logical(u, 16) & 1
    u2 = (u + 0x7FFF + lsb) & jnp.int32(-65536)
    return lax.bitcast_convert_type(u2, jnp.float32)


def _precompute(orig_ref, sq_ref, rnd_ref, m2_ref):
    def body(c, _):
        sl = pl.ds(c * L, L)
        v0 = orig_ref[0, sl]
        v1 = orig_ref[1, sl]
        v2 = orig_ref[2, sl]
        sq_ref[sl] = v0 * v0 + v1 * v1 + v2 * v2
        r0 = _round_bf16(v0)
        r1 = _round_bf16(v1)
        r2 = _round_bf16(v2)
        rnd_ref[0, sl] = r0
        rnd_ref[1, sl] = r1
        rnd_ref[2, sl] = r2
        m2_ref[0, sl] = -2.0 * r0
        m2_ref[1, sl] = -2.0 * r1
        m2_ref[2, sl] = -2.0 * r2
        return 0
    lax.fori_loop(0, N // L, body, 0)


def _min_sum_dir(own_b_ref, own_sq_ref, own_base, opp_m2_ref, opp_sq_ref):
    def group_body(gg, psum):
        base_a = own_base + gg * (2 * L)
        base_b = base_a + L
        a0 = own_b_ref[0, pl.ds(base_a, L)]
        a1 = own_b_ref[1, pl.ds(base_a, L)]
        a2 = own_b_ref[2, pl.ds(base_a, L)]
        b0 = own_b_ref[0, pl.ds(base_b, L)]
        b1 = own_b_ref[1, pl.ds(base_b, L)]
        b2 = own_b_ref[2, pl.ds(base_b, L)]

        def opp_body(c, accs):
            acc_a, acc_b = accs
            w0 = opp_m2_ref[0, pl.ds(c * L, L)]
            w1 = opp_m2_ref[1, pl.ds(c * L, L)]
            w2 = opp_m2_ref[2, pl.ds(c * L, L)]
            ws = opp_sq_ref[pl.ds(c * L, L)]
            for t in range(L):
                c0 = w0[t]
                c1 = w1[t]
                c2 = w2[t]
                cs = ws[t]
                e_a = cs + (a0 * c0 + a1 * c1 + a2 * c2)
                e_b = cs + (b0 * c0 + b1 * c1 + b2 * c2)
                acc_a = jnp.minimum(acc_a, e_a)
                acc_b = jnp.minimum(acc_b, e_b)
            return acc_a, acc_b

        init = (jnp.full((L,), 1e30, jnp.float32),
                jnp.full((L,), 1e30, jnp.float32))
        m_a, m_b = lax.fori_loop(0, NC, opp_body, init)
        sq_a = own_sq_ref[pl.ds(base_a, L)]
        sq_b = own_sq_ref[pl.ds(base_b, L)]
        zero = jnp.zeros((L,), jnp.float32)
        return (psum + jnp.maximum(m_a + sq_a, zero)
                + jnp.maximum(m_b + sq_b, zero))

    return lax.fori_loop(0, NG // 2, group_body, jnp.zeros((L,), jnp.float32))


def _sc_body(x_hbm, y_hbm, out_hbm,
             xorig, yorig, xb, yb, xm2, ym2, xsq, ysq, obuf):
    c = lax.axis_index("c")
    s = lax.axis_index("s")
    wid = s * 2 + c
    b = wid // WPB
    base = (wid % WPB) * CH

    pltpu.sync_copy(x_hbm.at[b], xorig)
    pltpu.sync_copy(y_hbm.at[b], yorig)

    _precompute(xorig, xsq, xb, xm2)
    _precompute(yorig, ysq, yb, ym2)

    px = _min_sum_dir(xb, xsq, base, ym2, ysq)
    py = _min_sum_dir(yb, ysq, base, xm2, xsq)
    obuf[...] = px + py
    pltpu.sync_copy(obuf, out_hbm.at[wid])


def kernel(xyz1, xyz2):
    xT = jnp.transpose(xyz1, (0, 2, 1))
    yT = jnp.transpose(xyz2, (0, 2, 1))
    mesh = plsc.VectorSubcoreMesh(core_axis_name="c", subcore_axis_name="s")
    out = pl.kernel(
        _sc_body,
        out_type=jax.ShapeDtypeStruct((NW, L), jnp.float32),
        mesh=mesh,
        scratch_types=[
            pltpu.VMEM((3, N), jnp.float32),
            pltpu.VMEM((3, M), jnp.float32),
            pltpu.VMEM((3, N), jnp.float32),
            pltpu.VMEM((3, M), jnp.float32),
            pltpu.VMEM((3, N), jnp.float32),
            pltpu.VMEM((3, M), jnp.float32),
            pltpu.VMEM((N,), jnp.float32),
            pltpu.VMEM((M,), jnp.float32),
            pltpu.VMEM((L,), jnp.float32),
        ],
    )(xT, yT)
    return jnp.sum(out) * (1.0 / (B * N))


# speedup vs baseline: 14.5676x; 14.5676x over previous
import jax
import jax.numpy as jnp
from jax import lax
from jax.experimental import pallas as pl
from jax.experimental.pallas import tpu as pltpu

B, N, M = 4, 4096, 4096
TN = 4096
NT = N // TN


def _limbs(v):
    h = v.astype(jnp.bfloat16)
    r1 = v - h.astype(jnp.float32)
    m = r1.astype(jnp.bfloat16)
    l = (r1 - m.astype(jnp.float32)).astype(jnp.bfloat16)
    return h, m, l


def _chamfer_body(x_ref, y_ref, out_ref, ymin_ref, xsum_ref):
    i = pl.program_id(1)
    b = pl.program_id(0)

    x = x_ref[0]
    y = y_ref[0]
    xb = x.astype(jnp.bfloat16)
    w = (y.astype(jnp.bfloat16)) * jnp.bfloat16(-2.0)

    xsq = jnp.sum(x * x, axis=1, keepdims=True)
    ysq = jnp.sum(y * y, axis=0, keepdims=True)
    xh, xm, xl = _limbs(xsq)
    yh, ym, yl = _limbs(ysq)

    ones_x = jnp.ones((TN, 3), jnp.bfloat16)
    ones_y = jnp.ones((3, M), jnp.bfloat16)
    xmat = jnp.concatenate([xb, ones_x, xh, xm, xl], axis=1)
    wmat = jnp.concatenate([w, yh, ym, yl, ones_y], axis=0)

    g = jax.lax.dot_general(xmat, wmat, (((1,), (0,)), ((), ())),
                            preferred_element_type=jnp.float32)

    row_min = jnp.maximum(jnp.min(g, axis=1), 0.0)
    col_min = jnp.min(g, axis=0, keepdims=True)

    @pl.when(i == 0)
    def _init():
        ymin_ref[...] = col_min
        xsum_ref[0, 0] = jnp.sum(row_min)

    @pl.when(i > 0)
    def _acc():
        ymin_ref[...] = jnp.minimum(ymin_ref[...], col_min)
        xsum_ref[0, 0] = xsum_ref[0, 0] + jnp.sum(row_min)

    @pl.when(jnp.logical_and(b == 0, i == 0))
    def _zero_out():
        out_ref[...] = jnp.zeros((1, 1), jnp.float32)

    @pl.when(i == NT - 1)
    def _finish():
        total = xsum_ref[0, 0] + jnp.sum(
            jnp.maximum(ymin_ref[...], 0.0))
        out_ref[...] = out_ref[...] + total * (1.0 / (B * N))


def kernel(xyz1, xyz2):
    yT = jnp.transpose(xyz2, (0, 2, 1))
    out = pl.pallas_call(
        _chamfer_body,
        grid=(B, NT),
        in_specs=[
            pl.BlockSpec((1, TN, 3), lambda b, i: (b, i, 0)),
            pl.BlockSpec((1, 3, M), lambda b, i: (b, 0, 0)),
        ],
        out_specs=pl.BlockSpec((1, 1), lambda b, i: (0, 0)),
        out_shape=jax.ShapeDtypeStruct((1, 1), jnp.float32),
        scratch_shapes=[
            pltpu.VMEM((1, M), jnp.float32),
            pltpu.SMEM((1, 1), jnp.float32),
        ],
    )(xyz1, yT)
    return out[0, 0]
